# baseline (speedup 1.0000x reference)
import jax
import jax.numpy as jnp
from jax import lax
from jax.experimental import pallas as pl
from jax.experimental.pallas import tpu as pltpu
from jax.experimental.pallas import tpu_sc as plsc

_M, _D, _B = 1000000, 32, 16384
_NT = 16
_NB = _B // _NT
_NG = _NB // 16


def _body(mem_hbm, idx_hbm, val_hbm,
          idx_v, pos_v, val_v, old_v, psum_v, t_v, w_v, wch_v, row_v,
          tbl_hbm, brd_hbm, wbd_hbm, sem):
    s = lax.axis_index("s")
    base = s * jnp.int32(_NB)

    pltpu.sync_copy(idx_hbm.at[pl.ds(base, _NB)], idx_v)
    pltpu.sync_copy(val_hbm.at[pl.ds(base, _NB)], val_v)

    def genpos(g, c):
        gb = g * jnp.int32(16)
        pos_v[pl.ds(gb, 16)] = base + gb + lax.iota(jnp.int32, 16)
        return c
    lax.fori_loop(jnp.int32(0), jnp.int32(_NG), genpos, jnp.int32(0))

    pltpu.sync_copy(pos_v, tbl_hbm.at[idx_v])

    pltpu.async_copy(mem_hbm.at[idx_v], old_v, sem).wait()

    def grp(g, c):
        rb = g * jnp.int32(16)
        for r in range(16):
            row = rb + r
            a0 = old_v[row, pl.ds(0, 16)]
            a1 = old_v[row, pl.ds(16, 16)]
            b0 = val_v[row, pl.ds(0, 16)]
            b1 = val_v[row, pl.ds(16, 16)]
            psum_v[row, pl.ds(0, 16)] = a0 * b0 + a1 * b1
        rows16 = rb + lax.iota(jnp.int32, 16)
        acc = jnp.zeros((16,), jnp.float32)
        for dcol in range(16):
            cols = jnp.full((16,), dcol, jnp.int32)
            acc = acc + plsc.load_gather(psum_v, [rows16, cols])
        sgn = jnp.sign(acc)
        e2 = jnp.exp(-2.0 * jnp.abs(acc))
        t_v[pl.ds(rb, 16)] = sgn * (1.0 - 2.0 * e2 / (1.0 + e2))
        return c
    lax.fori_loop(jnp.int32(0), jnp.int32(_NG), grp, jnp.int32(0))

    def rowfn(g, c):
        rb = g * jnp.int32(16)
        t16 = t_v[pl.ds(rb, 16)]
        for r in range(16):
            row = rb + r
            tt = t16[r]
            u0 = val_v[row, pl.ds(0, 16)] * tt
            u1 = val_v[row, pl.ds(16, 16)] * tt
            val_v[row, pl.ds(0, 16)] = u0
            val_v[row, pl.ds(16, 16)] = u1
            old_v[row, pl.ds(0, 16)] += u0
            old_v[row, pl.ds(16, 16)] += u1
        return c
    lax.fori_loop(jnp.int32(0), jnp.int32(_NG), rowfn, jnp.int32(0))

    pltpu.sync_copy(val_v, brd_hbm.at[pl.ds(base, _NB)])
    plsc.subcore_barrier()

    pltpu.async_copy(tbl_hbm.at[idx_v], w_v, sem).wait()
    pltpu.sync_copy(w_v, wbd_hbm.at[pl.ds(base, _NB)])
    plsc.subcore_barrier()

    lo = base
    hi = base + jnp.int32(_NB)
    lane = lax.iota(jnp.int32, 16)

    def scan_slab(k, c):
        kb = k * jnp.int32(_NB)
        pltpu.sync_copy(wbd_hbm.at[pl.ds(kb, _NB)], wch_v)

        def scan_chunk(q, c2):
            qb = q * jnp.int32(16)
            w16 = wch_v[pl.ds(qb, 16)]
            jvec = kb + qb + lane
            m = (w16 >= lo) & (w16 < hi) & (w16 != jvec)
            hits = plsc.all_reduce_population_count(m)

            @pl.when(hits[0] > 0)
            def _():
                mi = m.astype(jnp.int32)
                for l in range(16):
                    ml = mi[l]

                    @pl.when(ml > 0)
                    def _():
                        j = kb + qb + jnp.int32(l)
                        wj = w16[l]
                        lr = wj - lo
                        pltpu.sync_copy(brd_hbm.at[pl.ds(j, 1)], row_v)
                        old_v[lr, pl.ds(0, 16)] += row_v[0, pl.ds(0, 16)]
                        old_v[lr, pl.ds(16, 16)] += row_v[0, pl.ds(16, 16)]
            return c2
        lax.fori_loop(jnp.int32(0), jnp.int32(_NG), scan_chunk, jnp.int32(0))
        return c
    lax.fori_loop(jnp.int32(0), jnp.int32(_NT), scan_slab, jnp.int32(0))

    plsc.subcore_barrier()
    pltpu.sync_copy(old_v, brd_hbm.at[pl.ds(base, _NB)])
    plsc.subcore_barrier()

    pltpu.async_copy(brd_hbm.at[w_v], val_v, sem).wait()
    pltpu.sync_copy(val_v, mem_hbm.at[idx_v])


_mesh = plsc.VectorSubcoreMesh(
    core_axis_name="c", subcore_axis_name="s", num_cores=1)

_sc_update = pl.kernel(
    _body,
    out_type=(),
    mesh=_mesh,
    scratch_types=[
        pltpu.VMEM((_NB,), jnp.int32),
        pltpu.VMEM((_NB,), jnp.int32),
        pltpu.VMEM((_NB, _D), jnp.float32),
        pltpu.VMEM((_NB, _D), jnp.float32),
        pltpu.VMEM((_NB, 16), jnp.float32),
        pltpu.VMEM((_NB,), jnp.float32),
        pltpu.VMEM((_NB,), jnp.int32),
        pltpu.VMEM((_NB,), jnp.int32),
        pltpu.VMEM((1, _D), jnp.float32),
        pltpu.HBM((_M,), jnp.int32),
        pltpu.HBM((_B, _D), jnp.float32),
        pltpu.HBM((_B,), jnp.int32),
        pltpu.SemaphoreType.DMA,
    ],
    compiler_params=pltpu.CompilerParams(
        needs_layout_passes=False, use_tc_tiling_on_sc=False),
)


def kernel(mem, idx, val):
    idx32 = idx.astype(jnp.int32)
    mem_ref = jax.new_ref(mem)
    _sc_update(mem_ref, idx32, val)
    return mem_ref[...]
